# Initial kernel scaffold; baseline (speedup 1.0000x reference)
import functools

import jax
import jax.numpy as jnp
from jax import lax
from jax.experimental import pallas as pl
from jax.experimental.pallas import tpu as pltpu
from jax.experimental.pallas import tpu_sc as plsc

N = 10000
E = 320000
D = 128

NPAD = 10240
NC = 2
NS = 16
NW = NC * NS
CH = 79
EPT = CH * 128
EPAD = NW * EPT
ROWS_PER_TILE = NPAD // NS

_mesh = plsc.VectorSubcoreMesh(core_axis_name="c", subcore_axis_name="s")


def _zero_vec_buf(buf, n):
    z16 = jnp.zeros((16,), jnp.float32)

    def body(k, carry):
        buf[pl.ds(k * 16, 16)] = z16
        return carry

    lax.fori_loop(0, n // 16, body, 0)


def _zero_row_buf(buf):
    z16 = jnp.zeros((16,), jnp.float32)

    def body(k, carry):
        r = k // 8
        c = (k % 8) * 16
        buf[r, pl.ds(c, 16)] = z16
        return carry

    lax.fori_loop(0, 128 * 8, body, 0)


@functools.partial(
    pl.kernel,
    out_type=jax.ShapeDtypeStruct((NC, NS, ROWS_PER_TILE), jnp.float32),
    mesh=_mesh,
    scratch_types=[
        pltpu.VMEM((CH, 128), jnp.int32),
        pltpu.VMEM((128,), jnp.float32),
        pltpu.VMEM((ROWS_PER_TILE,), jnp.float32),
        pltpu.VMEM_SHARED((NPAD,), jnp.float32),
    ],
    name="gcn_deg_sc",
)
def _deg_kernel(dst_hbm, out_hbm, dst_v, ones_v, zeros_v, deg_sh):
    cid = lax.axis_index("c")
    sid = lax.axis_index("s")
    wid = cid * NS + sid

    pltpu.sync_copy(dst_hbm.at[wid], dst_v)
    one16 = jnp.ones((16,), jnp.float32)
    for i in range(8):
        ones_v[pl.ds(i * 16, 16)] = one16
    _zero_vec_buf(zeros_v, ROWS_PER_TILE)
    pltpu.sync_copy(zeros_v, deg_sh.at[pl.ds(sid * ROWS_PER_TILE, ROWS_PER_TILE)])
    plsc.subcore_barrier()

    def body(j, carry):
        pltpu.sync_copy(ones_v, deg_sh.at[dst_v.at[j]], add=True)
        return carry

    lax.fori_loop(0, CH, body, 0)
    plsc.subcore_barrier()
    pltpu.sync_copy(deg_sh.at[pl.ds(sid * ROWS_PER_TILE, ROWS_PER_TILE)],
                    out_hbm.at[cid, sid])


@functools.partial(
    pl.kernel,
    out_type=jax.ShapeDtypeStruct((NC, NS, ROWS_PER_TILE, D), jnp.float32),
    mesh=_mesh,
    scratch_types=[
        pltpu.VMEM((CH, 128), jnp.int32),
        pltpu.VMEM((CH, 128), jnp.int32),
        pltpu.VMEM((128, D), jnp.float32),
        pltpu.VMEM((128, D), jnp.float32),
        pltpu.VMEM_SHARED((NPAD, D), jnp.float32),
        pltpu.SemaphoreType.DMA,
        pltpu.SemaphoreType.DMA,
    ],
    name="gcn_edge_sc",
)
def _edge_kernel(src_hbm, dst_hbm, g_hbm, out_hbm,
                 src_v, dst_v, rows0, rows1, acc_sh, sem0, sem1):
    cid = lax.axis_index("c")
    sid = lax.axis_index("s")
    wid = cid * NS + sid

    pltpu.sync_copy(src_hbm.at[wid], src_v)
    pltpu.sync_copy(dst_hbm.at[wid], dst_v)

    _zero_row_buf(rows0)
    base = sid * ROWS_PER_TILE
    for b in range(ROWS_PER_TILE // 128):
        pltpu.sync_copy(rows0, acc_sh.at[pl.ds(base + b * 128, 128)])
    plsc.subcore_barrier()

    pltpu.async_copy(g_hbm.at[src_v.at[0]], rows0, sem0).wait()

    def body(j, carry):
        nxt = pltpu.async_copy(g_hbm.at[src_v.at[j + 1]], rows1, sem1)
        pltpu.sync_copy(rows0, acc_sh.at[dst_v.at[j]], add=True)
        nxt.wait()
        nxt2 = pltpu.async_copy(g_hbm.at[src_v.at[j + 2]], rows0, sem0)
        pltpu.sync_copy(rows1, acc_sh.at[dst_v.at[j + 1]], add=True)
        nxt2.wait()
        return carry

    lax.fori_loop(0, (CH - 1) // 2, lambda i, c: body(i * 2, c), 0)
    pltpu.sync_copy(rows0, acc_sh.at[dst_v.at[CH - 1]], add=True)

    plsc.subcore_barrier()
    for b in range(ROWS_PER_TILE // 128):
        pltpu.sync_copy(acc_sh.at[pl.ds(base + b * 128, 128)],
                        out_hbm.at[cid, sid].at[pl.ds(b * 128, 128)])


@functools.partial(
    pl.kernel,
    out_type=jax.ShapeDtypeStruct((NC, NS, ROWS_PER_TILE), jnp.float32),
    mesh=_mesh,
    scratch_types=[
        pltpu.VMEM((CH, 128), jnp.int32),
        pltpu.VMEM((CH, 128), jnp.int32),
        pltpu.VMEM((128,), jnp.float32),
        pltpu.VMEM((128,), jnp.float32),
        pltpu.VMEM((ROWS_PER_TILE,), jnp.float32),
        pltpu.VMEM_SHARED((NPAD,), jnp.float32),
        pltpu.SemaphoreType.DMA,
        pltpu.SemaphoreType.DMA,
    ],
    name="gcn_edge2_sc",
)
def _edge2_kernel(src_hbm, dst_hbm, u_hbm, out_hbm,
                  src_v, dst_v, val0, val1, zeros_v, acc_sh, sem0, sem1):
    cid = lax.axis_index("c")
    sid = lax.axis_index("s")
    wid = cid * NS + sid

    pltpu.sync_copy(src_hbm.at[wid], src_v)
    pltpu.sync_copy(dst_hbm.at[wid], dst_v)
    _zero_vec_buf(zeros_v, ROWS_PER_TILE)
    pltpu.sync_copy(zeros_v, acc_sh.at[pl.ds(sid * ROWS_PER_TILE, ROWS_PER_TILE)])
    plsc.subcore_barrier()

    pltpu.async_copy(u_hbm.at[src_v.at[0]], val0, sem0).wait()

    def body(j, carry):
        nxt = pltpu.async_copy(u_hbm.at[src_v.at[j + 1]], val1, sem1)
        pltpu.sync_copy(val0, acc_sh.at[dst_v.at[j]], add=True)
        nxt.wait()
        nxt2 = pltpu.async_copy(u_hbm.at[src_v.at[j + 2]], val0, sem0)
        pltpu.sync_copy(val1, acc_sh.at[dst_v.at[j + 1]], add=True)
        nxt2.wait()
        return carry

    lax.fori_loop(0, (CH - 1) // 2, lambda i, c: body(i * 2, c), 0)
    pltpu.sync_copy(val0, acc_sh.at[dst_v.at[CH - 1]], add=True)

    plsc.subcore_barrier()
    pltpu.sync_copy(acc_sh.at[pl.ds(sid * ROWS_PER_TILE, ROWS_PER_TILE)],
                    out_hbm.at[cid, sid])


_BR = 512


def _dense1_body(x_ref, degp_ref, w1_ref, g_ref, dinv_ref):
    deg = degp_ref[0] + degp_ref[1] + 1.0
    dinv = lax.rsqrt(deg)
    h = jnp.dot(x_ref[...], w1_ref[...], preferred_element_type=jnp.float32)
    g_ref[...] = h * dinv
    dinv_ref[...] = dinv


def _dense2_body(accp_ref, g_ref, dinv_ref, b1_ref, w2_ref, u_ref):
    acc = accp_ref[0] + accp_ref[1] + g_ref[...]
    dinv = dinv_ref[...]
    out1 = acc * dinv + b1_ref[...]
    r = jnp.maximum(out1, 0.0)
    z = jnp.sum(r * w2_ref[...], axis=1, keepdims=True)
    u_ref[...] = z * dinv


def _final_body(saccp_ref, u_ref, dinv_ref, b2_ref, out_ref):
    s = saccp_ref[0] + saccp_ref[1] + u_ref[...]
    out_ref[...] = s * dinv_ref[...] + b2_ref[0, 0]


def kernel(x, edge_index, W1, b1, W2, b2):
    src = edge_index[0]
    dst = edge_index[1]
    npad_e = EPAD - E
    dummy = (N + (jnp.arange(npad_e, dtype=jnp.int32) % (NPAD - N))).astype(jnp.int32)
    src3 = jnp.concatenate([src, dummy]).reshape(NW, CH, 128)
    dst3 = jnp.concatenate([dst, dummy]).reshape(NW, CH, 128)
    x_pad = jnp.zeros((NPAD, D), x.dtype).at[:N].set(x)

    degp = _deg_kernel(dst3)
    degp3 = degp.reshape(NC, NPAD, 1)

    nblk = NPAD // _BR
    g, dinv = pl.pallas_call(
        _dense1_body,
        grid=(nblk,),
        in_specs=[
            pl.BlockSpec((_BR, D), lambda i: (i, 0)),
            pl.BlockSpec((NC, _BR, 1), lambda i: (0, i, 0)),
            pl.BlockSpec((D, D), lambda i: (0, 0)),
        ],
        out_specs=[
            pl.BlockSpec((_BR, D), lambda i: (i, 0)),
            pl.BlockSpec((_BR, 1), lambda i: (i, 0)),
        ],
        out_shape=[
            jax.ShapeDtypeStruct((NPAD, D), jnp.float32),
            jax.ShapeDtypeStruct((NPAD, 1), jnp.float32),
        ],
        name="gcn_dense1_tc",
    )(x_pad, degp3, W1)

    accp = _edge_kernel(src3, dst3, g)
    accp3 = accp.reshape(NC, NPAD, D)

    u = pl.pallas_call(
        _dense2_body,
        grid=(nblk,),
        in_specs=[
            pl.BlockSpec((NC, _BR, D), lambda i: (0, i, 0)),
            pl.BlockSpec((_BR, D), lambda i: (i, 0)),
            pl.BlockSpec((_BR, 1), lambda i: (i, 0)),
            pl.BlockSpec((1, D), lambda i: (0, 0)),
            pl.BlockSpec((1, D), lambda i: (0, 0)),
        ],
        out_specs=pl.BlockSpec((_BR, 1), lambda i: (i, 0)),
        out_shape=jax.ShapeDtypeStruct((NPAD, 1), jnp.float32),
        name="gcn_dense2_tc",
    )(accp3, g, dinv, b1.reshape(1, D), W2.reshape(1, D))

    u1d = u.reshape(NPAD)
    saccp = _edge2_kernel(src3, dst3, u1d)

    out = pl.pallas_call(
        _final_body,
        in_specs=[
            pl.BlockSpec((NC, NPAD // 128, 128), lambda: (0, 0, 0)),
            pl.BlockSpec((NPAD // 128, 128), lambda: (0, 0)),
            pl.BlockSpec((NPAD // 128, 128), lambda: (0, 0)),
            pl.BlockSpec((1, 1), lambda: (0, 0)),
        ],
        out_specs=pl.BlockSpec((NPAD // 128, 128), lambda: (0, 0)),
        out_shape=jax.ShapeDtypeStruct((NPAD // 128, 128), jnp.float32),
        name="gcn_final_tc",
    )(saccp.reshape(NC, NPAD // 128, 128),
      u.reshape(NPAD // 128, 128),
      dinv.reshape(NPAD // 128, 128),
      b2.reshape(1, 1))

    return out.reshape(NPAD)[:N]


# speedup vs baseline: 30.5133x; 30.5133x over previous
import functools

import jax
import jax.numpy as jnp
from jax import lax
from jax.experimental import pallas as pl
from jax.experimental.pallas import tpu as pltpu
from jax.experimental.pallas import tpu_sc as plsc

N = 10000
E = 320000
D = 128

NPAD = 10240
NC = 2
NS = 16
NW = NC * NS
EPAD = 323584
CH32 = EPAD // NW // 128
ROWS_PER_TILE = NPAD // NS


@functools.cache
def _sc_mesh():
    return plsc.VectorSubcoreMesh(
        core_axis_name="c", subcore_axis_name="s", num_cores=NC, num_subcores=NS)


def _zero_vec_buf(buf, n):
    z16 = jnp.zeros((16,), jnp.float32)

    def body(k, carry):
        buf[pl.ds(k * 16, 16)] = z16
        return carry

    lax.fori_loop(0, n // 16, body, 0)


def _zero_row_buf(buf, rows, cols):
    z16 = jnp.zeros((16,), jnp.float32)
    cs = cols // 16

    def body(k, carry):
        r = k // cs
        c = (k % cs) * 16
        buf[r, pl.ds(c, 16)] = z16
        return carry

    lax.fori_loop(0, rows * cs, body, 0)


def _pipelined_edge_loop(nchunks, src_v, dst_v, table_hbm, buf0, buf1,
                         acc_sh, sem0, sem1):
    pltpu.async_copy(table_hbm.at[src_v.at[0]], buf0, sem0).wait()

    def pair(j, carry):
        nxt = pltpu.async_copy(table_hbm.at[src_v.at[j + 1]], buf1, sem1)
        pltpu.sync_copy(buf0, acc_sh.at[dst_v.at[j]], add=True)
        nxt.wait()
        nxt2 = pltpu.async_copy(table_hbm.at[src_v.at[j + 2]], buf0, sem0)
        pltpu.sync_copy(buf1, acc_sh.at[dst_v.at[j + 1]], add=True)
        nxt2.wait()
        return carry

    npairs = (nchunks - 1) // 2
    lax.fori_loop(0, npairs, lambda i, c: pair(i * 2, c), 0)
    rem = 2 * npairs
    if nchunks % 2 == 1:
        pltpu.sync_copy(buf0, acc_sh.at[dst_v.at[rem]], add=True)
    else:
        nxt = pltpu.async_copy(table_hbm.at[src_v.at[rem + 1]], buf1, sem1)
        pltpu.sync_copy(buf0, acc_sh.at[dst_v.at[rem]], add=True)
        nxt.wait()
        pltpu.sync_copy(buf1, acc_sh.at[dst_v.at[rem + 1]], add=True)


def _deg_body(dst_hbm, out_hbm, dst_v, ones_v, zeros_v, deg_sh):
    cid = lax.axis_index("c")
    sid = lax.axis_index("s")
    wid = cid * NS + sid

    pltpu.sync_copy(dst_hbm.at[wid], dst_v)
    one16 = jnp.ones((16,), jnp.float32)
    for i in range(8):
        ones_v[pl.ds(i * 16, 16)] = one16
    _zero_vec_buf(zeros_v, ROWS_PER_TILE)
    pltpu.sync_copy(zeros_v, deg_sh.at[pl.ds(sid * ROWS_PER_TILE, ROWS_PER_TILE)])
    plsc.subcore_barrier()

    def body(j, carry):
        pltpu.sync_copy(ones_v, deg_sh.at[dst_v.at[j]], add=True)
        return carry

    lax.fori_loop(0, CH32, body, 0)
    plsc.subcore_barrier()
    pltpu.sync_copy(deg_sh.at[pl.ds(sid * ROWS_PER_TILE, ROWS_PER_TILE)],
                    out_hbm.at[cid, sid])


def _edge_body(src_hbm, dst_hbm, g_hbm, out_hbm,
               src_v, dst_v, rows0, acc_sh, sem0):
    cid = lax.axis_index("c")
    sid = lax.axis_index("s")
    wid = cid * NS + sid

    pltpu.sync_copy(src_hbm.at[wid], src_v)
    pltpu.sync_copy(dst_hbm.at[wid], dst_v)

    _zero_row_buf(rows0, 128, D)
    base = sid * ROWS_PER_TILE
    for b in range(ROWS_PER_TILE // 128):
        pltpu.sync_copy(rows0, acc_sh.at[pl.ds(base + b * 128, 128)])
    plsc.subcore_barrier()

    def body(j, carry):
        pltpu.async_copy(g_hbm.at[src_v.at[j]], rows0, sem0).wait()
        pltpu.sync_copy(rows0, acc_sh.at[dst_v.at[j]], add=True)
        return carry

    lax.fori_loop(0, CH32, body, 0)

    plsc.subcore_barrier()
    for b in range(ROWS_PER_TILE // 128):
        pltpu.sync_copy(acc_sh.at[pl.ds(base + b * 128, 128)],
                        out_hbm.at[cid, sid].at[pl.ds(b * 128, 128)])


def _edge2_body(src_hbm, dst_hbm, u_hbm, out_hbm,
                src_v, dst_v, val0, val1, zeros_v, acc_sh, sem0, sem1):
    cid = lax.axis_index("c")
    sid = lax.axis_index("s")
    wid = cid * NS + sid

    pltpu.sync_copy(src_hbm.at[wid], src_v)
    pltpu.sync_copy(dst_hbm.at[wid], dst_v)
    _zero_vec_buf(zeros_v, ROWS_PER_TILE)
    pltpu.sync_copy(zeros_v, acc_sh.at[pl.ds(sid * ROWS_PER_TILE, ROWS_PER_TILE)])
    plsc.subcore_barrier()

    _pipelined_edge_loop(CH32, src_v, dst_v, u_hbm, val0, val1,
                         acc_sh, sem0, sem1)

    plsc.subcore_barrier()
    pltpu.sync_copy(acc_sh.at[pl.ds(sid * ROWS_PER_TILE, ROWS_PER_TILE)],
                    out_hbm.at[cid, sid])


@functools.cache
def _sc_fns():
    mesh = _sc_mesh()
    deg = pl.kernel(
        _deg_body,
        out_type=jax.ShapeDtypeStruct((NC, NS, ROWS_PER_TILE), jnp.float32),
        mesh=mesh,
        scratch_types=[
            pltpu.VMEM((CH32, 128), jnp.int32),
            pltpu.VMEM((128,), jnp.float32),
            pltpu.VMEM((ROWS_PER_TILE,), jnp.float32),
            pltpu.VMEM_SHARED((NPAD,), jnp.float32),
        ],
        name="gcn_deg_sc",
    )
    edge = pl.kernel(
        _edge_body,
        out_type=jax.ShapeDtypeStruct((NC, NS, ROWS_PER_TILE, D), jnp.float32),
        mesh=mesh,
        scratch_types=[
            pltpu.VMEM((CH32, 128), jnp.int32),
            pltpu.VMEM((CH32, 128), jnp.int32),
            pltpu.VMEM((128, D), jnp.float32),
            pltpu.VMEM_SHARED((NPAD, D), jnp.float32),
            pltpu.SemaphoreType.DMA,
        ],
        name="gcn_edge_sc",
    )
    edge2 = pl.kernel(
        _edge2_body,
        out_type=jax.ShapeDtypeStruct((NC, NS, ROWS_PER_TILE), jnp.float32),
        mesh=mesh,
        scratch_types=[
            pltpu.VMEM((CH32, 128), jnp.int32),
            pltpu.VMEM((CH32, 128), jnp.int32),
            pltpu.VMEM((128,), jnp.float32),
            pltpu.VMEM((128,), jnp.float32),
            pltpu.VMEM((ROWS_PER_TILE,), jnp.float32),
            pltpu.VMEM_SHARED((NPAD,), jnp.float32),
            pltpu.SemaphoreType.DMA,
            pltpu.SemaphoreType.DMA,
        ],
        name="gcn_edge2_sc",
    )
    return deg, edge, edge2


_BR = 512


def _dense1_body(x_ref, degp_ref, w1_ref, g_ref, dinv_ref):
    deg = degp_ref[0] + degp_ref[1] + 1.0
    dinv = lax.rsqrt(deg)
    h = jnp.dot(x_ref[...], w1_ref[...], preferred_element_type=jnp.float32)
    g_ref[...] = h * dinv
    dinv_ref[...] = dinv


def _dense2_body(accp_ref, g_ref, dinv_ref, b1_ref, w2_ref, u_ref):
    acc = accp_ref[0] + accp_ref[1] + g_ref[...]
    dinv = dinv_ref[...]
    out1 = acc * dinv + b1_ref[...]
    r = jnp.maximum(out1, 0.0)
    z = jnp.sum(r * w2_ref[...], axis=1, keepdims=True)
    u_ref[...] = z * dinv


def _final_body(saccp_ref, u_ref, dinv_ref, b2_ref, out_ref):
    s = saccp_ref[0] + saccp_ref[1] + u_ref[...]
    out_ref[...] = s * dinv_ref[...] + b2_ref[0, 0]


def kernel(x, edge_index, W1, b1, W2, b2):
    src = edge_index[0]
    dst = edge_index[1]
    npad_e = EPAD - E
    dummy = (N + (jnp.arange(npad_e, dtype=jnp.int32) % (NPAD - N))).astype(jnp.int32)
    src_p = jnp.concatenate([src, dummy])
    dst_p = jnp.concatenate([dst, dummy])
    src3 = src_p.reshape(NW, CH32, 128)
    dst3 = dst_p.reshape(NW, CH32, 128)
    x_pad = jnp.zeros((NPAD, D), x.dtype).at[:N].set(x)

    deg_fn, edge_fn, edge2_fn = _sc_fns()

    degp = deg_fn(dst3)
    degp3 = degp.reshape(NC, NPAD, 1)

    nblk = NPAD // _BR
    g, dinv = pl.pallas_call(
        _dense1_body,
        grid=(nblk,),
        in_specs=[
            pl.BlockSpec((_BR, D), lambda i: (i, 0)),
            pl.BlockSpec((NC, _BR, 1), lambda i: (0, i, 0)),
            pl.BlockSpec((D, D), lambda i: (0, 0)),
        ],
        out_specs=[
            pl.BlockSpec((_BR, D), lambda i: (i, 0)),
            pl.BlockSpec((_BR, 1), lambda i: (i, 0)),
        ],
        out_shape=[
            jax.ShapeDtypeStruct((NPAD, D), jnp.float32),
            jax.ShapeDtypeStruct((NPAD, 1), jnp.float32),
        ],
        name="gcn_dense1_tc",
    )(x_pad, degp3, W1)

    accp = edge_fn(src3, dst3, g)
    accp3 = accp.reshape(NC, NPAD, D)

    u = pl.pallas_call(
        _dense2_body,
        grid=(nblk,),
        in_specs=[
            pl.BlockSpec((NC, _BR, D), lambda i: (0, i, 0)),
            pl.BlockSpec((_BR, D), lambda i: (i, 0)),
            pl.BlockSpec((_BR, 1), lambda i: (i, 0)),
            pl.BlockSpec((1, D), lambda i: (0, 0)),
            pl.BlockSpec((1, D), lambda i: (0, 0)),
        ],
        out_specs=pl.BlockSpec((_BR, 1), lambda i: (i, 0)),
        out_shape=jax.ShapeDtypeStruct((NPAD, 1), jnp.float32),
        name="gcn_dense2_tc",
    )(accp3, g, dinv, b1.reshape(1, D), W2.reshape(1, D))

    u1d = u.reshape(NPAD)
    saccp = edge2_fn(src3, dst3, u1d)

    out = pl.pallas_call(
        _final_body,
        in_specs=[
            pl.BlockSpec((NC, NPAD // 128, 128), lambda: (0, 0, 0)),
            pl.BlockSpec((NPAD // 128, 128), lambda: (0, 0)),
            pl.BlockSpec((NPAD // 128, 128), lambda: (0, 0)),
            pl.BlockSpec((1, 1), lambda: (0, 0)),
        ],
        out_specs=pl.BlockSpec((NPAD // 128, 128), lambda: (0, 0)),
        out_shape=jax.ShapeDtypeStruct((NPAD // 128, 128), jnp.float32),
        name="gcn_final_tc",
    )(saccp.reshape(NC, NPAD // 128, 128),
      u.reshape(NPAD // 128, 128),
      dinv.reshape(NPAD // 128, 128),
      b2.reshape(1, 1))

    return out.reshape(NPAD)[:N]


# speedup vs baseline: 37.2763x; 1.2216x over previous
import functools

import jax
import jax.numpy as jnp
from jax import lax
from jax.experimental import pallas as pl
from jax.experimental.pallas import tpu as pltpu
from jax.experimental.pallas import tpu_sc as plsc

N = 10000
E = 320000
D = 128

NPAD = 10240
NC = 2
NS = 16
NW = NC * NS
EPAD = 327680
CH32 = EPAD // NW // 128
ROWS_PER_TILE = NPAD // NS


@functools.cache
def _sc_mesh():
    return plsc.VectorSubcoreMesh(
        core_axis_name="c", subcore_axis_name="s", num_cores=NC, num_subcores=NS)


def _zero_vec_buf(buf, n):
    z16 = jnp.zeros((16,), jnp.float32)

    def body(k, carry):
        buf[pl.ds(k * 16, 16)] = z16
        return carry

    lax.fori_loop(0, n // 16, body, 0)


def _zero_row_buf(buf, rows, cols):
    z16 = jnp.zeros((16,), jnp.float32)
    cs = cols // 16

    def body(k, carry):
        r = k // cs
        c = (k % cs) * 16
        buf[r, pl.ds(c, 16)] = z16
        return carry

    lax.fori_loop(0, rows * cs, body, 0)


def _pipelined_edge_loop(nchunks, src_v, dst_v, dst_off, table_hbm, buf0,
                         buf1, acc_sh, sem0, sem1):
    assert nchunks % 2 == 0
    pltpu.async_copy(table_hbm.at[src_v.at[0]], buf0, sem0).wait()

    def pair(j, carry):
        nxt = pltpu.async_copy(table_hbm.at[src_v.at[j + 1]], buf1, sem1)
        pltpu.sync_copy(buf0, acc_sh.at[dst_v.at[dst_off + j]], add=True)
        nxt.wait()
        nxt2 = pltpu.async_copy(table_hbm.at[src_v.at[j + 2]], buf0, sem0)
        pltpu.sync_copy(buf1, acc_sh.at[dst_v.at[dst_off + j + 1]], add=True)
        nxt2.wait()
        return carry

    npairs = nchunks // 2 - 1
    lax.fori_loop(0, npairs, lambda i, c: pair(i * 2, c), 0)
    rem = 2 * npairs
    nxt = pltpu.async_copy(table_hbm.at[src_v.at[rem + 1]], buf1, sem1)
    pltpu.sync_copy(buf0, acc_sh.at[dst_v.at[dst_off + rem]], add=True)
    nxt.wait()
    pltpu.sync_copy(buf1, acc_sh.at[dst_v.at[dst_off + rem + 1]], add=True)


def _deg_body(dst_hbm, out_hbm, dst_v, ones_v, zeros_v, deg_sh):
    cid = lax.axis_index("c")
    sid = lax.axis_index("s")
    wid = cid * NS + sid

    pltpu.sync_copy(dst_hbm.at[wid], dst_v)
    one16 = jnp.ones((16,), jnp.float32)
    for i in range(8):
        ones_v[pl.ds(i * 16, 16)] = one16
    _zero_vec_buf(zeros_v, ROWS_PER_TILE)
    pltpu.sync_copy(zeros_v, deg_sh.at[pl.ds(sid * ROWS_PER_TILE, ROWS_PER_TILE)])
    plsc.subcore_barrier()

    def body(j, carry):
        pltpu.sync_copy(ones_v, deg_sh.at[dst_v.at[j]], add=True)
        return carry

    lax.fori_loop(0, CH32, body, 0)
    plsc.subcore_barrier()
    pltpu.sync_copy(deg_sh.at[pl.ds(sid * ROWS_PER_TILE, ROWS_PER_TILE)],
                    out_hbm.at[cid, sid])


def _edge_body(src_hbm, dst_hbm, g_hbm, out_hbm,
               src_v, dst_v, rows0, rows1, acc_sh, sem0, sem1):
    cid = lax.axis_index("c")
    sid = lax.axis_index("s")
    wid = cid * NS + sid
    half = CH32 // 2

    pltpu.sync_copy(dst_hbm.at[wid], dst_v)
    pltpu.sync_copy(src_hbm.at[wid].at[pl.ds(0, half)], src_v)

    _zero_row_buf(rows0, 128, D)
    base = sid * ROWS_PER_TILE
    for b in range(ROWS_PER_TILE // 128):
        pltpu.sync_copy(rows0, acc_sh.at[pl.ds(base + b * 128, 128)])
    plsc.subcore_barrier()

    _pipelined_edge_loop(half, src_v, dst_v, 0, g_hbm, rows0, rows1,
                         acc_sh, sem0, sem1)
    pltpu.sync_copy(src_hbm.at[wid].at[pl.ds(half, half)], src_v)
    _pipelined_edge_loop(half, src_v, dst_v, half, g_hbm, rows0, rows1,
                         acc_sh, sem0, sem1)

    plsc.subcore_barrier()
    for b in range(ROWS_PER_TILE // 128):
        pltpu.sync_copy(acc_sh.at[pl.ds(base + b * 128, 128)],
                        out_hbm.at[cid, sid].at[pl.ds(b * 128, 128)])


_NB2 = 4


def _edge2_body(src_hbm, dst_hbm, u_hbm, out_hbm,
                src_v, dst_v, vals, zeros_v, acc_sh, *sems):
    cid = lax.axis_index("c")
    sid = lax.axis_index("s")
    wid = cid * NS + sid
    gsem = sems[:_NB2]
    ssem = sems[_NB2:]

    pltpu.sync_copy(src_hbm.at[wid], src_v)
    pltpu.sync_copy(dst_hbm.at[wid], dst_v)
    _zero_vec_buf(zeros_v, ROWS_PER_TILE)
    pltpu.sync_copy(zeros_v, acc_sh.at[pl.ds(sid * ROWS_PER_TILE, ROWS_PER_TILE)])
    plsc.subcore_barrier()

    def wait_g(s, j):
        pltpu.make_async_copy(u_hbm.at[src_v.at[j]], vals.at[s], gsem[s]).wait()

    def wait_s(s, j):
        pltpu.make_async_copy(vals.at[s], acc_sh.at[dst_v.at[j]], ssem[s]).wait()

    for s in range(_NB2):
        pltpu.async_copy(u_hbm.at[src_v.at[s]], vals.at[s], gsem[s])

    nrounds = CH32 // _NB2

    def ring(i, carry):
        for s in range(_NB2):
            j = i * _NB2 + s
            wait_g(s, j)
            pltpu.async_copy(vals.at[s], acc_sh.at[dst_v.at[j]], ssem[s],
                             add=True)
        for s in range(_NB2):
            j = i * _NB2 + s
            wait_s(s, j)
            pltpu.async_copy(u_hbm.at[src_v.at[j + _NB2]], vals.at[s], gsem[s])
        return carry

    lax.fori_loop(0, nrounds - 1, ring, 0)
    last = (nrounds - 1) * _NB2
    for s in range(_NB2):
        wait_g(s, last + s)
        pltpu.async_copy(vals.at[s], acc_sh.at[dst_v.at[last + s]], ssem[s],
                         add=True)
    for s in range(_NB2):
        wait_s(s, last + s)

    plsc.subcore_barrier()
    pltpu.sync_copy(acc_sh.at[pl.ds(sid * ROWS_PER_TILE, ROWS_PER_TILE)],
                    out_hbm.at[cid, sid])


@functools.cache
def _sc_fns():
    mesh = _sc_mesh()
    deg = pl.kernel(
        _deg_body,
        out_type=jax.ShapeDtypeStruct((NC, NS, ROWS_PER_TILE), jnp.float32),
        mesh=mesh,
        scratch_types=[
            pltpu.VMEM((CH32, 128), jnp.int32),
            pltpu.VMEM((128,), jnp.float32),
            pltpu.VMEM((ROWS_PER_TILE,), jnp.float32),
            pltpu.VMEM_SHARED((NPAD,), jnp.float32),
        ],
        name="gcn_deg_sc",
    )
    edge = pl.kernel(
        _edge_body,
        out_type=jax.ShapeDtypeStruct((NC, NS, ROWS_PER_TILE, D), jnp.float32),
        mesh=mesh,
        scratch_types=[
            pltpu.VMEM((CH32 // 2, 128), jnp.int32),
            pltpu.VMEM((CH32, 128), jnp.int32),
            pltpu.VMEM((128, D), jnp.float32),
            pltpu.VMEM((128, D), jnp.float32),
            pltpu.VMEM_SHARED((NPAD, D), jnp.float32),
            pltpu.SemaphoreType.DMA,
            pltpu.SemaphoreType.DMA,
        ],
        name="gcn_edge_sc",
    )
    edge2 = pl.kernel(
        _edge2_body,
        out_type=jax.ShapeDtypeStruct((NC, NS, ROWS_PER_TILE), jnp.float32),
        mesh=mesh,
        scratch_types=[
            pltpu.VMEM((CH32, 128), jnp.int32),
            pltpu.VMEM((CH32, 128), jnp.int32),
            pltpu.VMEM((_NB2, 128), jnp.float32),
            pltpu.VMEM((ROWS_PER_TILE,), jnp.float32),
            pltpu.VMEM_SHARED((NPAD,), jnp.float32),
        ] + [pltpu.SemaphoreType.DMA] * (2 * _NB2),
        name="gcn_edge2_sc",
    )
    return deg, edge, edge2


_BR = 512


def _dense1_body(x_ref, degp_ref, w1_ref, g_ref, dinv_ref):
    deg = degp_ref[0] + degp_ref[1] + 1.0
    dinv = lax.rsqrt(deg)
    h = jnp.dot(x_ref[...], w1_ref[...], preferred_element_type=jnp.float32)
    g_ref[...] = h * dinv
    dinv_ref[...] = dinv


def _dense2_body(accp_ref, g_ref, dinv_ref, b1_ref, w2_ref, u_ref):
    acc = accp_ref[0] + accp_ref[1] + g_ref[...]
    dinv = dinv_ref[...]
    out1 = acc * dinv + b1_ref[...]
    r = jnp.maximum(out1, 0.0)
    z = jnp.sum(r * w2_ref[...], axis=1, keepdims=True)
    u_ref[...] = z * dinv


def _final_body(saccp_ref, u_ref, dinv_ref, b2_ref, out_ref):
    s = saccp_ref[0] + saccp_ref[1] + u_ref[...]
    out_ref[...] = s * dinv_ref[...] + b2_ref[0, 0]


def kernel(x, edge_index, W1, b1, W2, b2):
    src = edge_index[0]
    dst = edge_index[1]
    npad_e = EPAD - E
    dummy = (N + (jnp.arange(npad_e, dtype=jnp.int32) % (NPAD - N))).astype(jnp.int32)
    src_p = jnp.concatenate([src, dummy])
    dst_p = jnp.concatenate([dst, dummy])
    src3 = src_p.reshape(NW, CH32, 128)
    dst3 = dst_p.reshape(NW, CH32, 128)
    x_pad = jnp.zeros((NPAD, D), x.dtype).at[:N].set(x)

    deg_fn, edge_fn, edge2_fn = _sc_fns()

    degp = deg_fn(dst3)
    degp3 = degp.reshape(NC, NPAD, 1)

    nblk = NPAD // _BR
    g, dinv = pl.pallas_call(
        _dense1_body,
        grid=(nblk,),
        in_specs=[
            pl.BlockSpec((_BR, D), lambda i: (i, 0)),
            pl.BlockSpec((NC, _BR, 1), lambda i: (0, i, 0)),
            pl.BlockSpec((D, D), lambda i: (0, 0)),
        ],
        out_specs=[
            pl.BlockSpec((_BR, D), lambda i: (i, 0)),
            pl.BlockSpec((_BR, 1), lambda i: (i, 0)),
        ],
        out_shape=[
            jax.ShapeDtypeStruct((NPAD, D), jnp.float32),
            jax.ShapeDtypeStruct((NPAD, 1), jnp.float32),
        ],
        name="gcn_dense1_tc",
    )(x_pad, degp3, W1)

    accp = edge_fn(src3, dst3, g)
    accp3 = accp.reshape(NC, NPAD, D)

    u = pl.pallas_call(
        _dense2_body,
        grid=(nblk,),
        in_specs=[
            pl.BlockSpec((NC, _BR, D), lambda i: (0, i, 0)),
            pl.BlockSpec((_BR, D), lambda i: (i, 0)),
            pl.BlockSpec((_BR, 1), lambda i: (i, 0)),
            pl.BlockSpec((1, D), lambda i: (0, 0)),
            pl.BlockSpec((1, D), lambda i: (0, 0)),
        ],
        out_specs=pl.BlockSpec((_BR, 1), lambda i: (i, 0)),
        out_shape=jax.ShapeDtypeStruct((NPAD, 1), jnp.float32),
        name="gcn_dense2_tc",
    )(accp3, g, dinv, b1.reshape(1, D), W2.reshape(1, D))

    u1d = u.reshape(NPAD)
    saccp = edge2_fn(src3, dst3, u1d)

    out = pl.pallas_call(
        _final_body,
        in_specs=[
            pl.BlockSpec((NC, NPAD // 128, 128), lambda: (0, 0, 0)),
            pl.BlockSpec((NPAD // 128, 128), lambda: (0, 0)),
            pl.BlockSpec((NPAD // 128, 128), lambda: (0, 0)),
            pl.BlockSpec((1, 1), lambda: (0, 0)),
        ],
        out_specs=pl.BlockSpec((NPAD // 128, 128), lambda: (0, 0)),
        out_shape=jax.ShapeDtypeStruct((NPAD // 128, 128), jnp.float32),
        name="gcn_final_tc",
    )(saccp.reshape(NC, NPAD // 128, 128),
      u.reshape(NPAD // 128, 128),
      dinv.reshape(NPAD // 128, 128),
      b2.reshape(1, 1))

    return out.reshape(NPAD)[:N]


# speedup vs baseline: 39.2461x; 1.0528x over previous
import functools

import jax
import jax.numpy as jnp
from jax import lax
from jax.experimental import pallas as pl
from jax.experimental.pallas import tpu as pltpu
from jax.experimental.pallas import tpu_sc as plsc

N = 10000
E = 320000
D = 128

NPAD = 10240
NC = 2
NS = 16
NW = NC * NS
EPAD = 327680
CH32 = EPAD // NW // 128
ROWS_PER_TILE = NPAD // NS


@functools.cache
def _sc_mesh():
    return plsc.VectorSubcoreMesh(
        core_axis_name="c", subcore_axis_name="s", num_cores=NC, num_subcores=NS)


def _zero_vec_buf(buf, n):
    z16 = jnp.zeros((16,), jnp.float32)

    def body(k, carry):
        buf[pl.ds(k * 16, 16)] = z16
        return carry

    lax.fori_loop(0, n // 16, body, 0)


def _zero_row_buf(buf, rows, cols):
    z16 = jnp.zeros((16,), jnp.float32)
    cs = cols // 16

    def body(k, carry):
        r = k // cs
        c = (k % cs) * 16
        buf[r, pl.ds(c, 16)] = z16
        return carry

    lax.fori_loop(0, rows * cs, body, 0)


def _pipelined_edge_loop(nchunks, src_v, dst_v, dst_off, table_hbm, buf0,
                         buf1, acc_sh, sem0, sem1):
    assert nchunks % 2 == 0
    pltpu.async_copy(table_hbm.at[src_v.at[0]], buf0, sem0).wait()

    def pair(j, carry):
        nxt = pltpu.async_copy(table_hbm.at[src_v.at[j + 1]], buf1, sem1)
        pltpu.sync_copy(buf0, acc_sh.at[dst_v.at[dst_off + j]], add=True)
        nxt.wait()
        nxt2 = pltpu.async_copy(table_hbm.at[src_v.at[j + 2]], buf0, sem0)
        pltpu.sync_copy(buf1, acc_sh.at[dst_v.at[dst_off + j + 1]], add=True)
        nxt2.wait()
        return carry

    npairs = nchunks // 2 - 1
    lax.fori_loop(0, npairs, lambda i, c: pair(i * 2, c), 0)
    rem = 2 * npairs
    nxt = pltpu.async_copy(table_hbm.at[src_v.at[rem + 1]], buf1, sem1)
    pltpu.sync_copy(buf0, acc_sh.at[dst_v.at[dst_off + rem]], add=True)
    nxt.wait()
    pltpu.sync_copy(buf1, acc_sh.at[dst_v.at[dst_off + rem + 1]], add=True)


def _deg_body(dst_hbm, out_hbm, dst_v, ones_v, zeros_v, deg_sh):
    cid = lax.axis_index("c")
    sid = lax.axis_index("s")
    wid = cid * NS + sid

    pltpu.sync_copy(dst_hbm.at[wid], dst_v)
    one16 = jnp.ones((16,), jnp.float32)
    for i in range(8):
        ones_v[pl.ds(i * 16, 16)] = one16
    _zero_vec_buf(zeros_v, ROWS_PER_TILE)
    pltpu.sync_copy(zeros_v, deg_sh.at[pl.ds(sid * ROWS_PER_TILE, ROWS_PER_TILE)])
    plsc.subcore_barrier()

    def body(j, carry):
        pltpu.sync_copy(ones_v, deg_sh.at[dst_v.at[j]], add=True)
        return carry

    lax.fori_loop(0, CH32, body, 0)
    plsc.subcore_barrier()
    pltpu.sync_copy(deg_sh.at[pl.ds(sid * ROWS_PER_TILE, ROWS_PER_TILE)],
                    out_hbm.at[cid, sid])


def _edge_body(src_hbm, dst_hbm, g_hbm, out_hbm,
               src_v, dst_v, rows0, rows1, acc_sh, sem0, sem1):
    cid = lax.axis_index("c")
    sid = lax.axis_index("s")
    wid = cid * NS + sid
    half = CH32 // 2

    pltpu.sync_copy(dst_hbm.at[wid], dst_v)
    pltpu.sync_copy(src_hbm.at[wid].at[pl.ds(0, half)], src_v)

    _zero_row_buf(rows0, 128, D)
    base = sid * ROWS_PER_TILE
    for b in range(ROWS_PER_TILE // 128):
        pltpu.sync_copy(rows0, acc_sh.at[pl.ds(base + b * 128, 128)])
    plsc.subcore_barrier()

    _pipelined_edge_loop(half, src_v, dst_v, 0, g_hbm, rows0, rows1,
                         acc_sh, sem0, sem1)
    pltpu.sync_copy(src_hbm.at[wid].at[pl.ds(half, half)], src_v)
    _pipelined_edge_loop(half, src_v, dst_v, half, g_hbm, rows0, rows1,
                         acc_sh, sem0, sem1)

    plsc.subcore_barrier()
    for b in range(ROWS_PER_TILE // 128):
        pltpu.sync_copy(acc_sh.at[pl.ds(base + b * 128, 128)],
                        out_hbm.at[cid, sid].at[pl.ds(b * 128, 128)])


_NB2 = 8


def _edge2_body(src_hbm, dst_hbm, u_hbm, out_hbm,
                src_v, dst_v, vals, zeros_v, acc_sh, *sems):
    cid = lax.axis_index("c")
    sid = lax.axis_index("s")
    wid = cid * NS + sid
    gsem = sems[:_NB2]
    ssem = sems[_NB2:]

    pltpu.sync_copy(src_hbm.at[wid], src_v)
    pltpu.sync_copy(dst_hbm.at[wid], dst_v)
    _zero_vec_buf(zeros_v, ROWS_PER_TILE)
    pltpu.sync_copy(zeros_v, acc_sh.at[pl.ds(sid * ROWS_PER_TILE, ROWS_PER_TILE)])
    plsc.subcore_barrier()

    def wait_g(s, j):
        pltpu.make_async_copy(u_hbm.at[src_v.at[j]], vals.at[s], gsem[s]).wait()

    def wait_s(s, j):
        pltpu.make_async_copy(vals.at[s], acc_sh.at[dst_v.at[j]], ssem[s]).wait()

    for s in range(_NB2):
        pltpu.async_copy(u_hbm.at[src_v.at[s]], vals.at[s], gsem[s])

    nrounds = CH32 // _NB2

    def ring(i, carry):
        for s in range(_NB2):
            j = i * _NB2 + s
            wait_g(s, j)
            pltpu.async_copy(vals.at[s], acc_sh.at[dst_v.at[j]], ssem[s],
                             add=True)
        for s in range(_NB2):
            j = i * _NB2 + s
            wait_s(s, j)
            pltpu.async_copy(u_hbm.at[src_v.at[j + _NB2]], vals.at[s], gsem[s])
        return carry

    lax.fori_loop(0, nrounds - 1, ring, 0)
    last = (nrounds - 1) * _NB2
    for s in range(_NB2):
        wait_g(s, last + s)
        pltpu.async_copy(vals.at[s], acc_sh.at[dst_v.at[last + s]], ssem[s],
                         add=True)
    for s in range(_NB2):
        wait_s(s, last + s)

    plsc.subcore_barrier()
    pltpu.sync_copy(acc_sh.at[pl.ds(sid * ROWS_PER_TILE, ROWS_PER_TILE)],
                    out_hbm.at[cid, sid])


@functools.cache
def _sc_fns():
    mesh = _sc_mesh()
    deg = pl.kernel(
        _deg_body,
        out_type=jax.ShapeDtypeStruct((NC, NS, ROWS_PER_TILE), jnp.float32),
        mesh=mesh,
        scratch_types=[
            pltpu.VMEM((CH32, 128), jnp.int32),
            pltpu.VMEM((128,), jnp.float32),
            pltpu.VMEM((ROWS_PER_TILE,), jnp.float32),
            pltpu.VMEM_SHARED((NPAD,), jnp.float32),
        ],
        name="gcn_deg_sc",
    )
    edge = pl.kernel(
        _edge_body,
        out_type=jax.ShapeDtypeStruct((NC, NS, ROWS_PER_TILE, D), jnp.float32),
        mesh=mesh,
        scratch_types=[
            pltpu.VMEM((CH32 // 2, 128), jnp.int32),
            pltpu.VMEM((CH32, 128), jnp.int32),
            pltpu.VMEM((128, D), jnp.float32),
            pltpu.VMEM((128, D), jnp.float32),
            pltpu.VMEM_SHARED((NPAD, D), jnp.float32),
            pltpu.SemaphoreType.DMA,
            pltpu.SemaphoreType.DMA,
        ],
        name="gcn_edge_sc",
    )
    edge2 = pl.kernel(
        _edge2_body,
        out_type=jax.ShapeDtypeStruct((NC, NS, ROWS_PER_TILE), jnp.float32),
        mesh=mesh,
        scratch_types=[
            pltpu.VMEM((CH32, 128), jnp.int32),
            pltpu.VMEM((CH32, 128), jnp.int32),
            pltpu.VMEM((_NB2, 128), jnp.float32),
            pltpu.VMEM((ROWS_PER_TILE,), jnp.float32),
            pltpu.VMEM_SHARED((NPAD,), jnp.float32),
        ] + [pltpu.SemaphoreType.DMA] * (2 * _NB2),
        name="gcn_edge2_sc",
    )
    return deg, edge, edge2


_BR = 512


def _mm1_body(x_ref, w1_ref, h_ref):
    h_ref[...] = jnp.dot(x_ref[...], w1_ref[...],
                         preferred_element_type=jnp.float32)


def _scale_body(h_ref, degp_ref, g_ref, dinv_ref):
    deg = degp_ref[0] + degp_ref[1] + 1.0
    dinv = lax.rsqrt(deg)
    g_ref[...] = h_ref[...] * dinv
    dinv_ref[...] = dinv


def _dense2_body(accp_ref, g_ref, dinv_ref, b1_ref, w2_ref, u_ref):
    acc = accp_ref[0] + accp_ref[1] + g_ref[...]
    dinv = dinv_ref[...]
    out1 = acc * dinv + b1_ref[...]
    r = jnp.maximum(out1, 0.0)
    z = jnp.sum(r * w2_ref[...], axis=1, keepdims=True)
    u_ref[...] = z * dinv


def _final_body(saccp_ref, u_ref, dinv_ref, b2_ref, out_ref):
    s = saccp_ref[0] + saccp_ref[1] + u_ref[...]
    out_ref[...] = s * dinv_ref[...] + b2_ref[0, 0]


def kernel(x, edge_index, W1, b1, W2, b2):
    src = edge_index[0]
    dst = edge_index[1]
    npad_e = EPAD - E
    dummy = (N + (jnp.arange(npad_e, dtype=jnp.int32) % (NPAD - N))).astype(jnp.int32)
    src_p = jnp.concatenate([src, dummy])
    dst_p = jnp.concatenate([dst, dummy])
    src3 = src_p.reshape(NW, CH32, 128)
    dst3 = dst_p.reshape(NW, CH32, 128)
    deg_fn, edge_fn, edge2_fn = _sc_fns()

    degp = deg_fn(dst3)
    degp3 = degp.reshape(NC, NPAD, 1)

    nblk = NPAD // _BR
    h = pl.pallas_call(
        _mm1_body,
        grid=(nblk,),
        in_specs=[
            pl.BlockSpec((_BR, D), lambda i: (i, 0)),
            pl.BlockSpec((D, D), lambda i: (0, 0)),
        ],
        out_specs=pl.BlockSpec((_BR, D), lambda i: (i, 0)),
        out_shape=jax.ShapeDtypeStruct((NPAD, D), jnp.float32),
        name="gcn_mm1_tc",
    )(x, W1)

    g, dinv = pl.pallas_call(
        _scale_body,
        grid=(nblk,),
        in_specs=[
            pl.BlockSpec((_BR, D), lambda i: (i, 0)),
            pl.BlockSpec((NC, _BR, 1), lambda i: (0, i, 0)),
        ],
        out_specs=[
            pl.BlockSpec((_BR, D), lambda i: (i, 0)),
            pl.BlockSpec((_BR, 1), lambda i: (i, 0)),
        ],
        out_shape=[
            jax.ShapeDtypeStruct((NPAD, D), jnp.float32),
            jax.ShapeDtypeStruct((NPAD, 1), jnp.float32),
        ],
        name="gcn_scale_tc",
    )(h, degp3)

    accp = edge_fn(src3, dst3, g)
    accp3 = accp.reshape(NC, NPAD, D)

    u = pl.pallas_call(
        _dense2_body,
        grid=(nblk,),
        in_specs=[
            pl.BlockSpec((NC, _BR, D), lambda i: (0, i, 0)),
            pl.BlockSpec((_BR, D), lambda i: (i, 0)),
            pl.BlockSpec((_BR, 1), lambda i: (i, 0)),
            pl.BlockSpec((1, D), lambda i: (0, 0)),
            pl.BlockSpec((1, D), lambda i: (0, 0)),
        ],
        out_specs=pl.BlockSpec((_BR, 1), lambda i: (i, 0)),
        out_shape=jax.ShapeDtypeStruct((NPAD, 1), jnp.float32),
        name="gcn_dense2_tc",
    )(accp3, g, dinv, b1.reshape(1, D), W2.reshape(1, D))

    u1d = u.reshape(NPAD)
    saccp = edge2_fn(src3, dst3, u1d)

    out = pl.pallas_call(
        _final_body,
        in_specs=[
            pl.BlockSpec((NC, NPAD // 128, 128), lambda: (0, 0, 0)),
            pl.BlockSpec((NPAD // 128, 128), lambda: (0, 0)),
            pl.BlockSpec((NPAD // 128, 128), lambda: (0, 0)),
            pl.BlockSpec((1, 1), lambda: (0, 0)),
        ],
        out_specs=pl.BlockSpec((NPAD // 128, 128), lambda: (0, 0)),
        out_shape=jax.ShapeDtypeStruct((NPAD // 128, 128), jnp.float32),
        name="gcn_final_tc",
    )(saccp.reshape(NC, NPAD // 128, 128),
      u.reshape(NPAD // 128, 128),
      dinv.reshape(NPAD // 128, 128),
      b2.reshape(1, 1))

    return out.reshape(NPAD)[:N]


# speedup vs baseline: 41.4200x; 1.0554x over previous
import functools

import jax
import jax.numpy as jnp
from jax import lax
from jax.experimental import pallas as pl
from jax.experimental.pallas import tpu as pltpu
from jax.experimental.pallas import tpu_sc as plsc

N = 10000
E = 320000
D = 128

NPAD = 10240
NC = 2
NS = 16
NW = NC * NS
EPAD = 327680
CH32 = EPAD // NW // 128
ROWS_PER_TILE = NPAD // NS


@functools.cache
def _sc_mesh():
    return plsc.VectorSubcoreMesh(
        core_axis_name="c", subcore_axis_name="s", num_cores=NC, num_subcores=NS)


def _zero_vec_buf(buf, n):
    z16 = jnp.zeros((16,), jnp.float32)

    def body(k, carry):
        buf[pl.ds(k * 16, 16)] = z16
        return carry

    lax.fori_loop(0, n // 16, body, 0)


def _zero_row_buf(buf, rows, cols):
    z16 = jnp.zeros((16,), jnp.float32)
    cs = cols // 16

    def body(k, carry):
        r = k // cs
        c = (k % cs) * 16
        buf[r, pl.ds(c, 16)] = z16
        return carry

    lax.fori_loop(0, rows * cs, body, 0)


def _pipelined_edge_loop(nchunks, src_v, dst_v, dst_off, table_hbm, buf0,
                         buf1, acc_sh, sem0, sem1):
    assert nchunks % 2 == 0
    pltpu.async_copy(table_hbm.at[src_v.at[0]], buf0, sem0).wait()

    def pair(j, carry):
        nxt = pltpu.async_copy(table_hbm.at[src_v.at[j + 1]], buf1, sem1)
        pltpu.sync_copy(buf0, acc_sh.at[dst_v.at[dst_off + j]], add=True)
        nxt.wait()
        nxt2 = pltpu.async_copy(table_hbm.at[src_v.at[j + 2]], buf0, sem0)
        pltpu.sync_copy(buf1, acc_sh.at[dst_v.at[dst_off + j + 1]], add=True)
        nxt2.wait()
        return carry

    npairs = nchunks // 2 - 1
    lax.fori_loop(0, npairs, lambda i, c: pair(i * 2, c), 0)
    rem = 2 * npairs
    nxt = pltpu.async_copy(table_hbm.at[src_v.at[rem + 1]], buf1, sem1)
    pltpu.sync_copy(buf0, acc_sh.at[dst_v.at[dst_off + rem]], add=True)
    nxt.wait()
    pltpu.sync_copy(buf1, acc_sh.at[dst_v.at[dst_off + rem + 1]], add=True)


def _deg_body(dst_hbm, out_hbm, dst_v, ones_v, zeros_v, deg_sh):
    cid = lax.axis_index("c")
    sid = lax.axis_index("s")
    wid = cid * NS + sid

    pltpu.sync_copy(dst_hbm.at[wid], dst_v)
    one16 = jnp.ones((16,), jnp.float32)
    for i in range(8):
        ones_v[pl.ds(i * 16, 16)] = one16
    _zero_vec_buf(zeros_v, ROWS_PER_TILE)
    pltpu.sync_copy(zeros_v, deg_sh.at[pl.ds(sid * ROWS_PER_TILE, ROWS_PER_TILE)])
    plsc.subcore_barrier()

    def body(j, carry):
        pltpu.sync_copy(ones_v, deg_sh.at[dst_v.at[j]], add=True)
        return carry

    lax.fori_loop(0, CH32, body, 0)
    plsc.subcore_barrier()
    pltpu.sync_copy(deg_sh.at[pl.ds(sid * ROWS_PER_TILE, ROWS_PER_TILE)],
                    out_hbm.at[cid, sid])


def _edge_body(src_hbm, dst_hbm, g_hbm, out_hbm,
               src_v, dst_v, rows0, rows1, acc_sh, sem0, sem1):
    cid = lax.axis_index("c")
    sid = lax.axis_index("s")
    wid = cid * NS + sid
    half = CH32 // 2

    pltpu.sync_copy(dst_hbm.at[wid], dst_v)
    pltpu.sync_copy(src_hbm.at[wid].at[pl.ds(0, half)], src_v)

    _zero_row_buf(rows0, 128, D)
    base = sid * ROWS_PER_TILE
    for b in range(ROWS_PER_TILE // 128):
        pltpu.sync_copy(rows0, acc_sh.at[pl.ds(base + b * 128, 128)])
    plsc.subcore_barrier()

    _pipelined_edge_loop(half, src_v, dst_v, 0, g_hbm, rows0, rows1,
                         acc_sh, sem0, sem1)
    pltpu.sync_copy(src_hbm.at[wid].at[pl.ds(half, half)], src_v)
    _pipelined_edge_loop(half, src_v, dst_v, half, g_hbm, rows0, rows1,
                         acc_sh, sem0, sem1)

    plsc.subcore_barrier()
    for b in range(ROWS_PER_TILE // 128):
        pltpu.sync_copy(acc_sh.at[pl.ds(base + b * 128, 128)],
                        out_hbm.at[cid, sid].at[pl.ds(b * 128, 128)])


_NB2 = 8


def _edge2_body(src_hbm, dst_hbm, u_hbm, out_hbm,
                src_v, dst_v, vals, zeros_v, acc_sh, *sems):
    cid = lax.axis_index("c")
    sid = lax.axis_index("s")
    wid = cid * NS + sid
    gsem = sems[:_NB2]
    ssem = sems[_NB2:]

    pltpu.sync_copy(src_hbm.at[wid], src_v)
    pltpu.sync_copy(dst_hbm.at[wid], dst_v)
    _zero_vec_buf(zeros_v, ROWS_PER_TILE)
    pltpu.sync_copy(zeros_v, acc_sh.at[pl.ds(sid * ROWS_PER_TILE, ROWS_PER_TILE)])
    plsc.subcore_barrier()

    def wait_g(s, j):
        pltpu.make_async_copy(u_hbm.at[src_v.at[j]], vals.at[s], gsem[s]).wait()

    def wait_s(s, j):
        pltpu.make_async_copy(vals.at[s], acc_sh.at[dst_v.at[j]], ssem[s]).wait()

    for s in range(_NB2):
        pltpu.async_copy(u_hbm.at[src_v.at[s]], vals.at[s], gsem[s])

    nrounds = CH32 // _NB2

    def ring(i, carry):
        for s in range(_NB2):
            j = i * _NB2 + s
            wait_g(s, j)
            pltpu.async_copy(vals.at[s], acc_sh.at[dst_v.at[j]], ssem[s],
                             add=True)
        for s in range(_NB2):
            j = i * _NB2 + s
            wait_s(s, j)
            pltpu.async_copy(u_hbm.at[src_v.at[j + _NB2]], vals.at[s], gsem[s])
        return carry

    lax.fori_loop(0, nrounds - 1, ring, 0)
    last = (nrounds - 1) * _NB2
    for s in range(_NB2):
        wait_g(s, last + s)
        pltpu.async_copy(vals.at[s], acc_sh.at[dst_v.at[last + s]], ssem[s],
                         add=True)
    for s in range(_NB2):
        wait_s(s, last + s)

    plsc.subcore_barrier()
    pltpu.sync_copy(acc_sh.at[pl.ds(sid * ROWS_PER_TILE, ROWS_PER_TILE)],
                    out_hbm.at[cid, sid])


@functools.cache
def _sc_fns():
    mesh = _sc_mesh()
    deg = pl.kernel(
        _deg_body,
        out_type=jax.ShapeDtypeStruct((NC, NS, ROWS_PER_TILE), jnp.float32),
        mesh=mesh,
        scratch_types=[
            pltpu.VMEM((CH32, 128), jnp.int32),
            pltpu.VMEM((128,), jnp.float32),
            pltpu.VMEM((ROWS_PER_TILE,), jnp.float32),
            pltpu.VMEM_SHARED((NPAD,), jnp.float32),
        ],
        name="gcn_deg_sc",
    )
    edge = pl.kernel(
        _edge_body,
        out_type=jax.ShapeDtypeStruct((NC, NS, ROWS_PER_TILE, D), jnp.float32),
        mesh=mesh,
        scratch_types=[
            pltpu.VMEM((CH32 // 2, 128), jnp.int32),
            pltpu.VMEM((CH32, 128), jnp.int32),
            pltpu.VMEM((128, D), jnp.float32),
            pltpu.VMEM((128, D), jnp.float32),
            pltpu.VMEM_SHARED((NPAD, D), jnp.float32),
            pltpu.SemaphoreType.DMA,
            pltpu.SemaphoreType.DMA,
        ],
        name="gcn_edge_sc",
    )
    edge2 = pl.kernel(
        _edge2_body,
        out_type=jax.ShapeDtypeStruct((NC, NS, ROWS_PER_TILE), jnp.float32),
        mesh=mesh,
        scratch_types=[
            pltpu.VMEM((CH32, 128), jnp.int32),
            pltpu.VMEM((CH32, 128), jnp.int32),
            pltpu.VMEM((_NB2, 128), jnp.float32),
            pltpu.VMEM((ROWS_PER_TILE,), jnp.float32),
            pltpu.VMEM_SHARED((NPAD,), jnp.float32),
        ] + [pltpu.SemaphoreType.DMA] * (2 * _NB2),
        name="gcn_edge2_sc",
    )
    return deg, edge, edge2


_BR = 512


def _mm1_body(x_ref, w1_ref, h_ref):
    h_ref[...] = jnp.dot(x_ref[...], w1_ref[...],
                         preferred_element_type=jnp.float32)


def _diag(dinv_row):
    ri = lax.broadcasted_iota(jnp.int32, (D, D), 0)
    ci = lax.broadcasted_iota(jnp.int32, (D, D), 1)
    return jnp.where(ri == ci, jnp.broadcast_to(dinv_row, (D, D)), 0.0)


def _scale_body(h_ref, degp_ref, g_ref, dinv_ref):
    deg = degp_ref[0] + degp_ref[1] + 1.0
    dinv = lax.rsqrt(deg)
    dinv_ref[...] = dinv
    for r in range(8):
        dr = dinv[r:r + 1, :]
        blk = h_ref[pl.ds(r * 128, 128), :]
        g_ref[pl.ds(r * 128, 128), :] = jnp.dot(
            _diag(dr), blk, preferred_element_type=jnp.float32)


def _dense2_body(accp_ref, g_ref, dinv_ref, b1_ref, w2_ref, u_ref):
    dinv = dinv_ref[...]
    for r in range(8):
        sl = pl.ds(r * 128, 128)
        acc = accp_ref[0, sl, :] + accp_ref[1, sl, :] + g_ref[sl, :]
        dr = dinv[r:r + 1, :]
        out1 = jnp.dot(_diag(dr), acc,
                       preferred_element_type=jnp.float32) + b1_ref[...]
        rr = jnp.maximum(out1, 0.0)
        z = lax.dot_general(w2_ref[...], rr, (((1,), (1,)), ((), ())),
                            preferred_element_type=jnp.float32)
        u_ref[pl.ds(r, 1), :] = z * dr


def _final_body(saccp_ref, u_ref, dinv_ref, b2_ref, out_ref):
    s = saccp_ref[0] + saccp_ref[1] + u_ref[...]
    out_ref[...] = s * dinv_ref[...] + b2_ref[0, 0]


def kernel(x, edge_index, W1, b1, W2, b2):
    src = edge_index[0]
    dst = edge_index[1]
    npad_e = EPAD - E
    dummy = (N + (jnp.arange(npad_e, dtype=jnp.int32) % (NPAD - N))).astype(jnp.int32)
    src_p = jnp.concatenate([src, dummy])
    dst_p = jnp.concatenate([dst, dummy])
    src3 = src_p.reshape(NW, CH32, 128)
    dst3 = dst_p.reshape(NW, CH32, 128)
    deg_fn, edge_fn, edge2_fn = _sc_fns()

    nrow = NPAD // 128

    degp = deg_fn(dst3)
    degp80 = degp.reshape(NC, nrow, 128)

    nblk = NPAD // _BR
    h = pl.pallas_call(
        _mm1_body,
        grid=(nblk,),
        in_specs=[
            pl.BlockSpec((_BR, D), lambda i: (i, 0)),
            pl.BlockSpec((D, D), lambda i: (0, 0)),
        ],
        out_specs=pl.BlockSpec((_BR, D), lambda i: (i, 0)),
        out_shape=jax.ShapeDtypeStruct((NPAD, D), jnp.float32),
        name="gcn_mm1_tc",
    )(x, W1)

    g, dinv = pl.pallas_call(
        _scale_body,
        grid=(nrow // 8,),
        in_specs=[
            pl.BlockSpec((1024, D), lambda i: (i, 0)),
            pl.BlockSpec((NC, 8, 128), lambda i: (0, i, 0)),
        ],
        out_specs=[
            pl.BlockSpec((1024, D), lambda i: (i, 0)),
            pl.BlockSpec((8, 128), lambda i: (i, 0)),
        ],
        out_shape=[
            jax.ShapeDtypeStruct((NPAD, D), jnp.float32),
            jax.ShapeDtypeStruct((nrow, 128), jnp.float32),
        ],
        name="gcn_scale_tc",
    )(h, degp80)

    accp = edge_fn(src3, dst3, g)
    accp3 = accp.reshape(NC, NPAD, D)

    u = pl.pallas_call(
        _dense2_body,
        grid=(nrow // 8,),
        in_specs=[
            pl.BlockSpec((NC, 1024, D), lambda i: (0, i, 0)),
            pl.BlockSpec((1024, D), lambda i: (i, 0)),
            pl.BlockSpec((8, 128), lambda i: (i, 0)),
            pl.BlockSpec((1, D), lambda i: (0, 0)),
            pl.BlockSpec((1, D), lambda i: (0, 0)),
        ],
        out_specs=pl.BlockSpec((8, 128), lambda i: (i, 0)),
        out_shape=jax.ShapeDtypeStruct((nrow, 128), jnp.float32),
        name="gcn_dense2_tc",
    )(accp3, g, dinv, b1.reshape(1, D), W2.reshape(1, D))

    u1d = u.reshape(NPAD)
    saccp = edge2_fn(src3, dst3, u1d)

    out = pl.pallas_call(
        _final_body,
        in_specs=[
            pl.BlockSpec((NC, nrow, 128), lambda: (0, 0, 0)),
            pl.BlockSpec((nrow, 128), lambda: (0, 0)),
            pl.BlockSpec((nrow, 128), lambda: (0, 0)),
            pl.BlockSpec((1, 1), lambda: (0, 0)),
        ],
        out_specs=pl.BlockSpec((nrow, 128), lambda: (0, 0)),
        out_shape=jax.ShapeDtypeStruct((nrow, 128), jnp.float32),
        name="gcn_final_tc",
    )(saccp.reshape(NC, nrow, 128), u, dinv, b2.reshape(1, 1))

    return out.reshape(NPAD)[:N]
